# baseline (speedup 1.0000x reference)
import functools

import jax
import jax.numpy as jnp
from jax import lax
from jax.experimental import pallas as pl
from jax.experimental.pallas import tpu as pltpu
from jax.experimental.pallas import tpu_sc as plsc

BATCH = 16384
SEQ_LEN = 200
EMBED_DIM = 64

_info = plsc.get_sparse_core_info()
_NC = _info.num_cores
_NS = _info.num_subcores
_NW = _NC * _NS
_NBLK = 128
_BLOCKS_PW = BATCH // (_NW * _NBLK)
_L = 16

_mesh = plsc.VectorSubcoreMesh(core_axis_name="c", subcore_axis_name="s")


@functools.partial(
    pl.kernel,
    mesh=_mesh,
    out_type=jax.ShapeDtypeStruct((SEQ_LEN, EMBED_DIM, BATCH), jnp.float32),
    scratch_types=[
        pltpu.VMEM((_NBLK, SEQ_LEN), jnp.int32),
        pltpu.VMEM((SEQ_LEN, _NBLK), jnp.int32),
        pltpu.VMEM((_NBLK, EMBED_DIM), jnp.float32),
        pltpu.VMEM((_NBLK, EMBED_DIM), jnp.float32),
        pltpu.VMEM((1, EMBED_DIM, _NBLK), jnp.float32),
        pltpu.VMEM((1, EMBED_DIM, _NBLK), jnp.float32),
        pltpu.SemaphoreType.DMA,
        pltpu.SemaphoreType.DMA,
        pltpu.SemaphoreType.DMA,
        pltpu.SemaphoreType.DMA,
        pltpu.SemaphoreType.DMA,
    ],
    compiler_params=pltpu.CompilerParams(
        use_tc_tiling_on_sc=False, needs_layout_passes=False),
)
def _gather_kernel(ids_hbm, table_hbm, y_hbm, idsblk, sidx, rows0, rows1,
                   slab0, slab1, s_ids, s_gat0, s_gat1, s_out0, s_out1):
    wid = lax.axis_index("s") * _NC + lax.axis_index("c")

    rows = (rows0, rows1)
    slab = (slab0, slab1)
    s_gat = (s_gat0, s_gat1)
    s_out = (s_out0, s_out1)

    lane = jax.lax.iota(jnp.int32, _L)
    row_idx = [lane + (_L * j) for j in range(_NBLK // _L)]

    def block(k, carry):
        b0 = (wid * _BLOCKS_PW + k) * _NBLK

        pltpu.async_copy(ids_hbm.at[pl.ds(b0, _NBLK), :], idsblk, s_ids)
        pltpu.make_async_copy(
            ids_hbm.at[pl.ds(b0, _NBLK), :], idsblk, s_ids).wait()

        def tr_ids(s, c):
            col = jnp.full((_L,), 0, jnp.int32) + s
            for j in range(_NBLK // _L):
                sidx[s, pl.ds(_L * j, _L)] = plsc.load_gather(
                    idsblk, [row_idx[j], col])
            return c
        lax.fori_loop(0, SEQ_LEN, tr_ids, 0)

        def gat_start(s, p):
            pltpu.async_copy(table_hbm.at[sidx.at[s]], rows[p], s_gat[p])

        def gat_wait(s, p):
            pltpu.make_async_copy(
                table_hbm.at[sidx.at[s]], rows[p], s_gat[p]).wait()

        def y_at(s):
            return y_hbm.at[pl.ds(s, 1), :, pl.ds(b0, _NBLK)]

        def transpose(p):
            def tr_e(e, c):
                col = jnp.full((_L,), 0, jnp.int32) + e
                for j in range(_NBLK // _L):
                    slab[p][0, e, pl.ds(_L * j, _L)] = plsc.load_gather(
                        rows[p], [row_idx[j], col])
                return c
            lax.fori_loop(0, EMBED_DIM, tr_e, 0)

        def half(s, p, *, first=False, last=False):
            gat_wait(s, p)
            if not last:
                gat_start(s + 1, 1 - p)
            if not first:
                pltpu.make_async_copy(slab[p], y_at(s - 2), s_out[p]).wait()
            transpose(p)
            pltpu.async_copy(slab[p], y_at(s), s_out[p])

        def spair(g, c):
            half(2 * g, 0)
            half(2 * g + 1, 1)
            return c

        gat_start(0, 0)
        half(0, 0, first=True)
        half(1, 1, first=True)
        lax.fori_loop(1, SEQ_LEN // 2 - 1, spair, 0)
        half(SEQ_LEN - 2, 0)
        half(SEQ_LEN - 1, 1, last=True)
        pltpu.make_async_copy(slab[0], y_at(SEQ_LEN - 2), s_out[0]).wait()
        pltpu.make_async_copy(slab[1], y_at(SEQ_LEN - 1), s_out[1]).wait()
        return carry

    lax.fori_loop(0, _BLOCKS_PW, block, 0)


def kernel(intent_ids, table):
    y = _gather_kernel(intent_ids.astype(jnp.int32), table)
    return jnp.transpose(y, (2, 0, 1))


# speedup vs baseline: 1.5924x; 1.5924x over previous
import functools

import jax
import jax.numpy as jnp
from jax import lax
from jax.experimental import pallas as pl
from jax.experimental.pallas import tpu as pltpu
from jax.experimental.pallas import tpu_sc as plsc

BATCH = 16384
SEQ_LEN = 200
EMBED_DIM = 64

_info = plsc.get_sparse_core_info()
_NC = _info.num_cores
_NS = _info.num_subcores
_NW = _NC * _NS
_NBLK = 128
_BLOCKS_PW = BATCH // (_NW * _NBLK)
_L = 16

_mesh = plsc.VectorSubcoreMesh(core_axis_name="c", subcore_axis_name="s")


@functools.partial(
    pl.kernel,
    mesh=_mesh,
    out_type=jax.ShapeDtypeStruct((SEQ_LEN, EMBED_DIM, BATCH), jnp.float32),
    scratch_types=[
        pltpu.VMEM((_NBLK, SEQ_LEN), jnp.int32),
        pltpu.VMEM((SEQ_LEN, _NBLK), jnp.int32),
        pltpu.VMEM((_NBLK, EMBED_DIM), jnp.float32),
        pltpu.VMEM((_NBLK, EMBED_DIM), jnp.float32),
        pltpu.VMEM((1, EMBED_DIM, _NBLK), jnp.float32),
        pltpu.VMEM((1, EMBED_DIM, _NBLK), jnp.float32),
        pltpu.SemaphoreType.DMA,
        pltpu.SemaphoreType.DMA,
        pltpu.SemaphoreType.DMA,
        pltpu.SemaphoreType.DMA,
        pltpu.SemaphoreType.DMA,
    ],
    compiler_params=pltpu.CompilerParams(
        use_tc_tiling_on_sc=False, needs_layout_passes=False),
)
def _gather_kernel(ids_hbm, table_hbm, y_hbm, idsblk, sidx, rows0, rows1,
                   slab0, slab1, s_ids, s_gat0, s_gat1, s_out0, s_out1):
    wid = lax.axis_index("s") * _NC + lax.axis_index("c")

    rows = (rows0, rows1)
    slab = (slab0, slab1)
    s_gat = (s_gat0, s_gat1)
    s_out = (s_out0, s_out1)

    lane = jax.lax.iota(jnp.int32, _L)
    row_idx = [lane + (_L * j) for j in range(_NBLK // _L)]

    def block(k, carry):
        b0 = (wid * _BLOCKS_PW + k) * _NBLK

        pltpu.async_copy(ids_hbm.at[pl.ds(b0, _NBLK), :], idsblk, s_ids)
        pltpu.make_async_copy(
            ids_hbm.at[pl.ds(b0, _NBLK), :], idsblk, s_ids).wait()

        @plsc.parallel_loop(0, SEQ_LEN, unroll=4)
        def tr_ids(s):
            col = jnp.full((_L,), 0, jnp.int32) + s
            for j in range(_NBLK // _L):
                sidx[s, pl.ds(_L * j, _L)] = plsc.load_gather(
                    idsblk, [row_idx[j], col])

        def gat_start(s, p):
            pltpu.async_copy(table_hbm.at[sidx.at[s]], rows[p], s_gat[p])

        def gat_wait(s, p):
            pltpu.make_async_copy(
                table_hbm.at[sidx.at[s]], rows[p], s_gat[p]).wait()

        def y_at(s):
            return y_hbm.at[pl.ds(s, 1), :, pl.ds(b0, _NBLK)]

        def transpose(p):
            @plsc.parallel_loop(0, EMBED_DIM, unroll=8)
            def tr_e(e):
                col = jnp.full((_L,), 0, jnp.int32) + e
                for j in range(_NBLK // _L):
                    slab[p][0, e, pl.ds(_L * j, _L)] = plsc.load_gather(
                        rows[p], [row_idx[j], col])

        def half(s, p, *, first=False, last=False):
            gat_wait(s, p)
            if not last:
                gat_start(s + 1, 1 - p)
            if not first:
                pltpu.make_async_copy(slab[p], y_at(s - 2), s_out[p]).wait()
            transpose(p)
            pltpu.async_copy(slab[p], y_at(s), s_out[p])

        def spair(g, c):
            half(2 * g, 0)
            half(2 * g + 1, 1)
            return c

        gat_start(0, 0)
        half(0, 0, first=True)
        half(1, 1, first=True)
        lax.fori_loop(1, SEQ_LEN // 2 - 1, spair, 0)
        half(SEQ_LEN - 2, 0)
        half(SEQ_LEN - 1, 1, last=True)
        pltpu.make_async_copy(slab[0], y_at(SEQ_LEN - 2), s_out[0]).wait()
        pltpu.make_async_copy(slab[1], y_at(SEQ_LEN - 1), s_out[1]).wait()
        return carry

    lax.fori_loop(0, _BLOCKS_PW, block, 0)


def kernel(intent_ids, table):
    y = _gather_kernel(intent_ids.astype(jnp.int32), table)
    return jnp.transpose(y, (2, 0, 1))
